# baseline (speedup 1.0000x reference)
import functools

import jax
import jax.numpy as jnp
from jax import lax
from jax.experimental import pallas as pl
from jax.experimental.pallas import tpu as pltpu
from jax.experimental.pallas import tpu_sc as plsc

N = 10000
E = 320000
F = 128
NEG = 0.2
BETA = 0.5
C = 1.2

NC = 2
NS = 16
NW = NC * NS
K = 128
NCHUNK = E // K
CH_SPAN = 80
CH_LAST = NCHUNK - (NW - 1) * CH_SPAN
NPAD = 10240
ACC_SPAN = 624
ACC_LAST = N - (NS - 1) * ACC_SPAN
ZR = 78
DEN_SPAN = NPAD // NS

RB = 400


def _prep_body(x_ref, w_ref, as_ref, ad_ref, xp_ref, a_ref, b_ref):
    xp = jnp.dot(x_ref[...], w_ref[...], preferred_element_type=jnp.float32)
    xp_ref[...] = xp
    a_ref[...] = jnp.dot(xp, as_ref[...], preferred_element_type=jnp.float32)
    b_ref[...] = jnp.dot(xp, ad_ref[...], preferred_element_type=jnp.float32)


def _prep(x, W, att_s, att_d):
    return pl.pallas_call(
        _prep_body,
        grid=(N // RB,),
        in_specs=[
            pl.BlockSpec((RB, F), lambda i: (i, 0)),
            pl.BlockSpec((F, F), lambda i: (0, 0)),
            pl.BlockSpec((F, 1), lambda i: (0, 0)),
            pl.BlockSpec((F, 1), lambda i: (0, 0)),
        ],
        out_specs=[
            pl.BlockSpec((RB, F), lambda i: (i, 0)),
            pl.BlockSpec((RB, 1), lambda i: (i, 0)),
            pl.BlockSpec((RB, 1), lambda i: (i, 0)),
        ],
        out_shape=[
            jax.ShapeDtypeStruct((N, F), jnp.float32),
            jax.ShapeDtypeStruct((N, 1), jnp.float32),
            jax.ShapeDtypeStruct((N, 1), jnp.float32),
        ],
    )(x, W, att_s, att_d)


def _edge_body(ei_hbm, asrc_hbm, adst_hbm, xp_hbm,
               acc_hbm, den_hbm,
               ei0, ei1, ei2, ei3,
               rows0, rows1, wv0, wv1, asg0, asg1, adg0, adg1, zbuf,
               acc_s, den_s, asv_s, adv_s,
               gsem0, gsem1, asem0, asem1, ssem0, ssem1, isem):
    c = lax.axis_index("c")
    s = lax.axis_index("s")
    wid = s * NC + c

    eib = (ei0, ei1, ei2, ei3)
    sidx = tuple(e.at[0] for e in eib)
    didx = tuple(e.at[1] for e in eib)
    rows = (rows0, rows1)
    wv = (wv0, wv1)
    asg = (asg0, asg1)
    adg = (adg0, adg1)
    gsem = (gsem0, gsem1)
    asem = (asem0, asem1)
    ssem = (ssem0, ssem1)

    @pl.when(s == 0)
    def _():
        pltpu.sync_copy(asrc_hbm, asv_s)
        pltpu.sync_copy(adst_hbm, adv_s)

    def _zb(i, _):
        r = i // (F // 16)
        j = i % (F // 16)
        zbuf[r, pl.ds(j * 16, 16)] = jnp.zeros((16,), jnp.float32)
        return 0
    lax.fori_loop(0, ZR * (F // 16), _zb, 0)

    for i in range(ACC_SPAN // ZR):
        pltpu.async_copy(zbuf, acc_s.at[pl.ds(s * ACC_SPAN + i * ZR, ZR)], isem)

    @pl.when(s == NS - 1)
    def _():
        pltpu.async_copy(zbuf.at[pl.ds(0, ACC_LAST - ACC_SPAN)],
                         acc_s.at[pl.ds(NS * ACC_SPAN, ACC_LAST - ACC_SPAN)], isem)

    for i in range(DEN_SPAN // F):
        pltpu.async_copy(zbuf.at[0], den_s.at[pl.ds(s * DEN_SPAN + i * F, F)], isem)

    for i in range(ACC_SPAN // ZR):
        pltpu.make_async_copy(zbuf, acc_s.at[pl.ds(s * ACC_SPAN + i * ZR, ZR)], isem).wait()

    @pl.when(s == NS - 1)
    def _():
        pltpu.make_async_copy(zbuf.at[pl.ds(0, ACC_LAST - ACC_SPAN)],
                              acc_s.at[pl.ds(NS * ACC_SPAN, ACC_LAST - ACC_SPAN)], isem).wait()

    for i in range(DEN_SPAN // F):
        pltpu.make_async_copy(zbuf.at[0], den_s.at[pl.ds(s * DEN_SPAN + i * F, F)], isem).wait()

    plsc.subcore_barrier()

    cstart = wid * CH_SPAN
    nch = jnp.where(wid < NW - 1, CH_SPAN, CH_LAST)

    def _idx_copy(j, ib):
        pltpu.async_copy(ei_hbm.at[cstart + j], eib[ib], isem)

    def _idx_wait(j, ib):
        pltpu.make_async_copy(ei_hbm.at[cstart + j], eib[ib], isem).wait()

    def _gathers(j, ib, b):
        pltpu.async_copy(asv_s.at[sidx[ib]], asg[b], asem[b])
        pltpu.async_copy(adv_s.at[didx[ib]], adg[b], asem[b])
        pltpu.async_copy(xp_hbm.at[sidx[ib]], rows[b], gsem[b])

    def _w_compute(ib, b):
        ab, bb, wb = asg[b], adg[b], wv[b]
        pltpu.make_async_copy(asv_s.at[sidx[ib]], ab, asem[b]).wait()
        pltpu.make_async_copy(adv_s.at[didx[ib]], bb, asem[b]).wait()

        @plsc.parallel_loop(0, K // 16, unroll=4)
        def _w(i):
            al = ab[pl.ds(i * 16, 16)] + bb[pl.ds(i * 16, 16)]
            al = jnp.where(al >= 0.0, al, al * NEG)
            wb[pl.ds(i * 16, 16)] = jnp.exp(al)

    def _scale(ib, b):
        rb, wb = rows[b], wv[b]
        pltpu.make_async_copy(xp_hbm.at[sidx[ib]], rb, gsem[b]).wait()

        @plsc.parallel_loop(0, K, unroll=16)
        def _sc(k):
            w16 = plsc.load_gather(wb, [jnp.zeros((16,), jnp.int32) + k])
            for i in range(F // 16):
                rb[k, pl.ds(i * 16, 16)] = rb[k, pl.ds(i * 16, 16)] * w16

    def _scatter(ib, b):
        pltpu.async_copy(rows[b], acc_s.at[didx[ib]], ssem[b], add=True)
        pltpu.async_copy(wv[b], den_s.at[didx[ib]], ssem[b], add=True)

    def _scatter_wait(ib, b):
        pltpu.make_async_copy(rows[b], acc_s.at[didx[ib]], ssem[b]).wait()
        pltpu.make_async_copy(wv[b], den_s.at[didx[ib]], ssem[b]).wait()

    def _iter(j, ib, b, first=False):
        b2 = 1 - b

        @pl.when(j + 1 < nch)
        def _():
            ib1 = (ib + 1) % 4
            _idx_wait(j + 1, ib1)
            if not first:
                _scatter_wait((ib + 3) % 4, b2)
            _gathers(j + 1, ib1, b2)

            @pl.when(j + 2 < nch)
            def _():
                _idx_copy(j + 2, (ib + 2) % 4)

        _w_compute(ib, b)
        _scale(ib, b)
        _scatter(ib, b)

    _idx_copy(jnp.int32(0), 0)
    _idx_wait(jnp.int32(0), 0)
    _gathers(jnp.int32(0), 0, 0)
    _idx_copy(jnp.int32(1), 1)

    _iter(jnp.int32(0), 0, 0, first=True)
    _iter(jnp.int32(1), 1, 1)
    _iter(jnp.int32(2), 2, 0)
    _iter(jnp.int32(3), 3, 1)

    def _quad(t, _):
        j = 4 * t
        _iter(j, 0, 0)
        _iter(j + 1, 1, 1)
        _iter(j + 2, 2, 0)
        _iter(j + 3, 3, 1)
        return 0
    lax.fori_loop(1, nch // 4, _quad, 0)

    _scatter_wait(2, 0)
    _scatter_wait(3, 1)

    plsc.subcore_barrier()

    @pl.when(s < NS - 1)
    def _():
        pltpu.sync_copy(acc_s.at[pl.ds(s * ACC_SPAN, ACC_SPAN)],
                        acc_hbm.at[c, pl.ds(s * ACC_SPAN, ACC_SPAN)])

    @pl.when(s == NS - 1)
    def _():
        pltpu.sync_copy(acc_s.at[pl.ds((NS - 1) * ACC_SPAN, ACC_LAST)],
                        acc_hbm.at[c, pl.ds((NS - 1) * ACC_SPAN, ACC_LAST)])

    pltpu.sync_copy(den_s.at[pl.ds(s * DEN_SPAN, DEN_SPAN)],
                    den_hbm.at[c, pl.ds(s * DEN_SPAN, DEN_SPAN)])


_edge = functools.partial(
    pl.kernel,
    out_type=[
        jax.ShapeDtypeStruct((NC, N, F), jnp.float32),
        jax.ShapeDtypeStruct((NC, NPAD), jnp.float32),
    ],
    mesh=plsc.VectorSubcoreMesh(core_axis_name="c", subcore_axis_name="s",
                                num_cores=NC, num_subcores=NS),
    compiler_params=pltpu.CompilerParams(needs_layout_passes=False),
    scratch_types=[
        pltpu.VMEM((2, K), jnp.int32),
        pltpu.VMEM((2, K), jnp.int32),
        pltpu.VMEM((2, K), jnp.int32),
        pltpu.VMEM((2, K), jnp.int32),
        pltpu.VMEM((K, F), jnp.float32),
        pltpu.VMEM((K, F), jnp.float32),
        pltpu.VMEM((K,), jnp.float32),
        pltpu.VMEM((K,), jnp.float32),
        pltpu.VMEM((K,), jnp.float32),
        pltpu.VMEM((K,), jnp.float32),
        pltpu.VMEM((K,), jnp.float32),
        pltpu.VMEM((K,), jnp.float32),
        pltpu.VMEM((ZR, F), jnp.float32),
        pltpu.VMEM_SHARED((N, F), jnp.float32),
        pltpu.VMEM_SHARED((NPAD,), jnp.float32),
        pltpu.VMEM_SHARED((NPAD,), jnp.float32),
        pltpu.VMEM_SHARED((NPAD,), jnp.float32),
        pltpu.SemaphoreType.DMA,
        pltpu.SemaphoreType.DMA,
        pltpu.SemaphoreType.DMA,
        pltpu.SemaphoreType.DMA,
        pltpu.SemaphoreType.DMA,
        pltpu.SemaphoreType.DMA,
        pltpu.SemaphoreType.DMA,
    ],
)(_edge_body)


def _post_body(acc_ref, den_ref, a_ref, b_ref, xp_ref, bias_ref, o_ref):
    acc = acc_ref[0] + acc_ref[1]
    den = den_ref[0] + den_ref[1]
    al = a_ref[...] + b_ref[...]
    al = jnp.where(al >= 0.0, al, al * NEG)
    ws = jnp.exp(al)
    num = acc + ws * xp_ref[...]
    d = den + ws + 1e-16
    z = num / d + bias_ref[...]
    o_ref[...] = BETA * z + (C - BETA) * (z * jax.nn.sigmoid(z))


def _post(acc, den, a, b, xp, bias):
    return pl.pallas_call(
        _post_body,
        grid=(N // RB,),
        in_specs=[
            pl.BlockSpec((NC, RB, F), lambda i: (0, i, 0)),
            pl.BlockSpec((NC, RB, 1), lambda i: (0, i, 0)),
            pl.BlockSpec((RB, 1), lambda i: (i, 0)),
            pl.BlockSpec((RB, 1), lambda i: (i, 0)),
            pl.BlockSpec((RB, F), lambda i: (i, 0)),
            pl.BlockSpec((1, F), lambda i: (0, 0)),
        ],
        out_specs=pl.BlockSpec((RB, F), lambda i: (i, 0)),
        out_shape=jax.ShapeDtypeStruct((N, F), jnp.float32),
    )(acc, den, a, b, xp, bias)


def kernel(x, edge_index, W, att_src, att_dst, bias):
    ei = jnp.stack([edge_index[0].astype(jnp.int32).reshape(NCHUNK, K),
                    edge_index[1].astype(jnp.int32).reshape(NCHUNK, K)], axis=1)
    att_s = att_src.reshape(F, 1)
    att_d = att_dst.reshape(F, 1)
    xp, a, b = _prep(x, W, att_s, att_d)
    apad = jnp.pad(a.reshape(N), (0, NPAD - N))
    bpad = jnp.pad(b.reshape(N), (0, NPAD - N))
    acc, den = _edge(ei, apad, bpad, xp)
    return _post(acc, den[:, :N].reshape(NC, N, 1), a, b, xp, bias.reshape(1, F))


# speedup vs baseline: 1.0090x; 1.0090x over previous
import functools

import jax
import jax.numpy as jnp
from jax import lax
from jax.experimental import pallas as pl
from jax.experimental.pallas import tpu as pltpu
from jax.experimental.pallas import tpu_sc as plsc

N = 10000
E = 320000
F = 128
NEG = 0.2
BETA = 0.5
C = 1.2

NC = 2
NS = 16
NW = NC * NS
K = 128
NCHUNK = E // K
CH_SPAN = 80
CH_LAST = NCHUNK - (NW - 1) * CH_SPAN
NPAD = 10240
ACC_SPAN = 624
ACC_LAST = N - (NS - 1) * ACC_SPAN
ZR = 78
DEN_SPAN = NPAD // NS

RB = 400


def _prep_body(x_ref, w_ref, as_ref, ad_ref, xp_ref, a_ref, b_ref):
    xp = jnp.dot(x_ref[...], w_ref[...], preferred_element_type=jnp.float32)
    xp_ref[...] = xp
    a_ref[...] = jnp.dot(xp, as_ref[...], preferred_element_type=jnp.float32)
    b_ref[...] = jnp.dot(xp, ad_ref[...], preferred_element_type=jnp.float32)


def _prep(x, W, att_s, att_d):
    return pl.pallas_call(
        _prep_body,
        grid=(N // RB,),
        in_specs=[
            pl.BlockSpec((RB, F), lambda i: (i, 0)),
            pl.BlockSpec((F, F), lambda i: (0, 0)),
            pl.BlockSpec((F, 1), lambda i: (0, 0)),
            pl.BlockSpec((F, 1), lambda i: (0, 0)),
        ],
        out_specs=[
            pl.BlockSpec((RB, F), lambda i: (i, 0)),
            pl.BlockSpec((RB, 1), lambda i: (i, 0)),
            pl.BlockSpec((RB, 1), lambda i: (i, 0)),
        ],
        out_shape=[
            jax.ShapeDtypeStruct((N, F), jnp.float32),
            jax.ShapeDtypeStruct((N, 1), jnp.float32),
            jax.ShapeDtypeStruct((N, 1), jnp.float32),
        ],
    )(x, W, att_s, att_d)


def _edge_body(ei_hbm, asrc_hbm, adst_hbm, xp_hbm,
               acc_hbm, den_hbm,
               ei0, ei1, ei2, ei3,
               rows0, rows1, wv0, wv1, asg0, asg1, adg0, adg1, zbuf,
               acc_s, den_s, asv_s, adv_s,
               gsem0, gsem1, asem0, asem1, ssem0, ssem1, isem):
    c = lax.axis_index("c")
    s = lax.axis_index("s")
    wid = s * NC + c

    eib = (ei0, ei1, ei2, ei3)
    sidx = tuple(e.at[0] for e in eib)
    didx = tuple(e.at[1] for e in eib)
    rows = (rows0, rows1)
    wv = (wv0, wv1)
    asg = (asg0, asg1)
    adg = (adg0, adg1)
    gsem = (gsem0, gsem1)
    asem = (asem0, asem1)
    ssem = (ssem0, ssem1)

    @pl.when(s == 0)
    def _():
        pltpu.sync_copy(asrc_hbm, asv_s)
        pltpu.sync_copy(adst_hbm, adv_s)

    def _zb(i, _):
        r = i // (F // 16)
        j = i % (F // 16)
        zbuf[r, pl.ds(j * 16, 16)] = jnp.zeros((16,), jnp.float32)
        return 0
    lax.fori_loop(0, ZR * (F // 16), _zb, 0)

    for i in range(ACC_SPAN // ZR):
        pltpu.async_copy(zbuf, acc_s.at[pl.ds(s * ACC_SPAN + i * ZR, ZR)], isem)

    @pl.when(s == NS - 1)
    def _():
        pltpu.async_copy(zbuf.at[pl.ds(0, ACC_LAST - ACC_SPAN)],
                         acc_s.at[pl.ds(NS * ACC_SPAN, ACC_LAST - ACC_SPAN)], isem)

    for i in range(DEN_SPAN // F):
        pltpu.async_copy(zbuf.at[0], den_s.at[pl.ds(s * DEN_SPAN + i * F, F)], isem)

    for i in range(ACC_SPAN // ZR):
        pltpu.make_async_copy(zbuf, acc_s.at[pl.ds(s * ACC_SPAN + i * ZR, ZR)], isem).wait()

    @pl.when(s == NS - 1)
    def _():
        pltpu.make_async_copy(zbuf.at[pl.ds(0, ACC_LAST - ACC_SPAN)],
                              acc_s.at[pl.ds(NS * ACC_SPAN, ACC_LAST - ACC_SPAN)], isem).wait()

    for i in range(DEN_SPAN // F):
        pltpu.make_async_copy(zbuf.at[0], den_s.at[pl.ds(s * DEN_SPAN + i * F, F)], isem).wait()

    plsc.subcore_barrier()

    cstart = wid * CH_SPAN
    nch = jnp.where(wid < NW - 1, CH_SPAN, CH_LAST)

    def _idx_copy(j, ib):
        pltpu.async_copy(ei_hbm.at[cstart + j], eib[ib], isem)

    def _idx_wait(j, ib):
        pltpu.make_async_copy(ei_hbm.at[cstart + j], eib[ib], isem).wait()

    def _gathers(j, ib, b):
        pltpu.async_copy(asv_s.at[sidx[ib]], asg[b], asem[b])
        pltpu.async_copy(adv_s.at[didx[ib]], adg[b], asem[b])
        pltpu.async_copy(xp_hbm.at[sidx[ib]], rows[b], gsem[b])

    def _w_compute(ib, b):
        ab, bb, wb = asg[b], adg[b], wv[b]
        pltpu.make_async_copy(asv_s.at[sidx[ib]], ab, asem[b]).wait()
        pltpu.make_async_copy(adv_s.at[didx[ib]], bb, asem[b]).wait()

        @plsc.parallel_loop(0, K // 16, unroll=2)
        def _w(i):
            al = ab[pl.ds(i * 16, 16)] + bb[pl.ds(i * 16, 16)]
            al = jnp.where(al >= 0.0, al, al * NEG)
            wb[pl.ds(i * 16, 16)] = jnp.exp(al)

    def _scale(ib, b):
        rb, wb = rows[b], wv[b]
        pltpu.make_async_copy(xp_hbm.at[sidx[ib]], rb, gsem[b]).wait()

        @plsc.parallel_loop(0, K, unroll=8)
        def _sc(k):
            w16 = plsc.load_gather(wb, [jnp.zeros((16,), jnp.int32) + k])
            for i in range(F // 16):
                rb[k, pl.ds(i * 16, 16)] = rb[k, pl.ds(i * 16, 16)] * w16

    def _scatter(ib, b):
        pltpu.async_copy(rows[b], acc_s.at[didx[ib]], ssem[b], add=True)
        pltpu.async_copy(wv[b], den_s.at[didx[ib]], ssem[b], add=True)

    def _scatter_wait(ib, b):
        pltpu.make_async_copy(rows[b], acc_s.at[didx[ib]], ssem[b]).wait()
        pltpu.make_async_copy(wv[b], den_s.at[didx[ib]], ssem[b]).wait()

    def _iter(j, ib, b, first=False):
        b2 = 1 - b

        @pl.when(j + 1 < nch)
        def _():
            ib1 = (ib + 1) % 4
            _idx_wait(j + 1, ib1)
            if not first:
                _scatter_wait((ib + 3) % 4, b2)
            _gathers(j + 1, ib1, b2)

            @pl.when(j + 2 < nch)
            def _():
                _idx_copy(j + 2, (ib + 2) % 4)

        _w_compute(ib, b)
        _scale(ib, b)
        _scatter(ib, b)

    _idx_copy(jnp.int32(0), 0)
    _idx_wait(jnp.int32(0), 0)
    _gathers(jnp.int32(0), 0, 0)
    _idx_copy(jnp.int32(1), 1)

    _iter(jnp.int32(0), 0, 0, first=True)
    _iter(jnp.int32(1), 1, 1)
    _iter(jnp.int32(2), 2, 0)
    _iter(jnp.int32(3), 3, 1)

    def _quad(t, _):
        j = 4 * t
        _iter(j, 0, 0)
        _iter(j + 1, 1, 1)
        _iter(j + 2, 2, 0)
        _iter(j + 3, 3, 1)
        return 0
    lax.fori_loop(1, nch // 4, _quad, 0)

    _scatter_wait(2, 0)
    _scatter_wait(3, 1)

    plsc.subcore_barrier()

    @pl.when(s < NS - 1)
    def _():
        pltpu.sync_copy(acc_s.at[pl.ds(s * ACC_SPAN, ACC_SPAN)],
                        acc_hbm.at[c, pl.ds(s * ACC_SPAN, ACC_SPAN)])

    @pl.when(s == NS - 1)
    def _():
        pltpu.sync_copy(acc_s.at[pl.ds((NS - 1) * ACC_SPAN, ACC_LAST)],
                        acc_hbm.at[c, pl.ds((NS - 1) * ACC_SPAN, ACC_LAST)])

    pltpu.sync_copy(den_s.at[pl.ds(s * DEN_SPAN, DEN_SPAN)],
                    den_hbm.at[c, pl.ds(s * DEN_SPAN, DEN_SPAN)])


_edge = functools.partial(
    pl.kernel,
    out_type=[
        jax.ShapeDtypeStruct((NC, N, F), jnp.float32),
        jax.ShapeDtypeStruct((NC, NPAD), jnp.float32),
    ],
    mesh=plsc.VectorSubcoreMesh(core_axis_name="c", subcore_axis_name="s",
                                num_cores=NC, num_subcores=NS),
    compiler_params=pltpu.CompilerParams(needs_layout_passes=False),
    scratch_types=[
        pltpu.VMEM((2, K), jnp.int32),
        pltpu.VMEM((2, K), jnp.int32),
        pltpu.VMEM((2, K), jnp.int32),
        pltpu.VMEM((2, K), jnp.int32),
        pltpu.VMEM((K, F), jnp.float32),
        pltpu.VMEM((K, F), jnp.float32),
        pltpu.VMEM((K,), jnp.float32),
        pltpu.VMEM((K,), jnp.float32),
        pltpu.VMEM((K,), jnp.float32),
        pltpu.VMEM((K,), jnp.float32),
        pltpu.VMEM((K,), jnp.float32),
        pltpu.VMEM((K,), jnp.float32),
        pltpu.VMEM((ZR, F), jnp.float32),
        pltpu.VMEM_SHARED((N, F), jnp.float32),
        pltpu.VMEM_SHARED((NPAD,), jnp.float32),
        pltpu.VMEM_SHARED((NPAD,), jnp.float32),
        pltpu.VMEM_SHARED((NPAD,), jnp.float32),
        pltpu.SemaphoreType.DMA,
        pltpu.SemaphoreType.DMA,
        pltpu.SemaphoreType.DMA,
        pltpu.SemaphoreType.DMA,
        pltpu.SemaphoreType.DMA,
        pltpu.SemaphoreType.DMA,
        pltpu.SemaphoreType.DMA,
    ],
)(_edge_body)


def _post_body(acc_ref, den_ref, a_ref, b_ref, xp_ref, bias_ref, o_ref):
    acc = acc_ref[0] + acc_ref[1]
    den = den_ref[0] + den_ref[1]
    al = a_ref[...] + b_ref[...]
    al = jnp.where(al >= 0.0, al, al * NEG)
    ws = jnp.exp(al)
    num = acc + ws * xp_ref[...]
    d = den + ws + 1e-16
    z = num / d + bias_ref[...]
    o_ref[...] = BETA * z + (C - BETA) * (z * jax.nn.sigmoid(z))


def _post(acc, den, a, b, xp, bias):
    return pl.pallas_call(
        _post_body,
        grid=(N // RB,),
        in_specs=[
            pl.BlockSpec((NC, RB, F), lambda i: (0, i, 0)),
            pl.BlockSpec((NC, RB, 1), lambda i: (0, i, 0)),
            pl.BlockSpec((RB, 1), lambda i: (i, 0)),
            pl.BlockSpec((RB, 1), lambda i: (i, 0)),
            pl.BlockSpec((RB, F), lambda i: (i, 0)),
            pl.BlockSpec((1, F), lambda i: (0, 0)),
        ],
        out_specs=pl.BlockSpec((RB, F), lambda i: (i, 0)),
        out_shape=jax.ShapeDtypeStruct((N, F), jnp.float32),
    )(acc, den, a, b, xp, bias)


def kernel(x, edge_index, W, att_src, att_dst, bias):
    ei = jnp.stack([edge_index[0].astype(jnp.int32).reshape(NCHUNK, K),
                    edge_index[1].astype(jnp.int32).reshape(NCHUNK, K)], axis=1)
    att_s = att_src.reshape(F, 1)
    att_d = att_dst.reshape(F, 1)
    xp, a, b = _prep(x, W, att_s, att_d)
    apad = jnp.pad(a.reshape(N), (0, NPAD - N))
    bpad = jnp.pad(b.reshape(N), (0, NPAD - N))
    acc, den = _edge(ei, apad, bpad, xp)
    return _post(acc, den[:, :N].reshape(NC, N, 1), a, b, xp, bias.reshape(1, F))
